# Initial kernel scaffold; baseline (speedup 1.0000x reference)
import jax
import jax.numpy as jnp
from jax.experimental import pallas as pl


def kernel(memory, nodes, memory_dim):
    raise NotImplementedError("write your pallas kernel here")


# speedup vs baseline: 1.3124x; 1.3124x over previous
import functools

import jax
import jax.numpy as jnp
from jax import lax
from jax.experimental import pallas as pl
from jax.experimental.pallas import tpu as pltpu
from jax.experimental.pallas import tpu_sc as plsc

_L = 16


@functools.lru_cache(maxsize=None)
def _build(B, V, D, half):
    NC, NS = 2, 16
    NW = NC * NS
    b_per_w = B // NW
    CH = 256
    n_ch = b_per_w // CH

    mesh = plsc.VectorSubcoreMesh(core_axis_name="c", subcore_axis_name="s")

    @functools.partial(
        pl.kernel,
        mesh=mesh,
        out_type=(
            jax.ShapeDtypeStruct((B, half), jnp.float32),
            jax.ShapeDtypeStruct((B, half), jnp.float32),
        ),
        scratch_types=[
            pltpu.VMEM((b_per_w,), jnp.int32),
            pltpu.VMEM((CH, D), jnp.float32),
            pltpu.VMEM((CH, half), jnp.float32),
            pltpu.VMEM((CH, half), jnp.float32),
            pltpu.SemaphoreType.DMA,
        ],
    )
    def k(mem_hbm, idx_hbm, emb_hbm, num_hbm, idx_v, rows_v, emb_v, num_v, sem):
        wid = lax.axis_index("s") * NC + lax.axis_index("c")
        base = wid * b_per_w
        pltpu.sync_copy(idx_hbm.at[pl.ds(base, b_per_w)], idx_v)
        for c in range(n_ch):
            cbase = c * CH
            pltpu.async_copy(
                mem_hbm.at[idx_v.at[pl.ds(cbase, CH)]], rows_v, sem
            ).wait()

            def body(i, _):
                for j in range(half // _L):
                    num = rows_v[i, pl.ds(j * _L, _L)]
                    den = rows_v[i, pl.ds(half + j * _L, _L)]
                    num_v[i, pl.ds(j * _L, _L)] = num
                    emb_v[i, pl.ds(j * _L, _L)] = num / den
                return 0

            lax.fori_loop(0, CH, body, 0)
            pltpu.sync_copy(emb_v, emb_hbm.at[pl.ds(base + cbase, CH)])
            pltpu.sync_copy(num_v, num_hbm.at[pl.ds(base + cbase, CH)])

    return k


def kernel(memory, nodes, memory_dim):
    V, D = memory.shape
    B = nodes.shape[0]
    half = D // 2
    k = _build(B, V, D, half)
    emb, num = k(memory, nodes.astype(jnp.int32))
    return (emb, num)
